# baseline (speedup 1.0000x reference)
import jax
import jax.numpy as jnp
from jax.experimental import pallas as pl
from jax.experimental.pallas import tpu as pltpu

_E_ROWS = 1_000_000
_DIM = 32
_WIDE_COLS = 256
_WIDE_ROWS = (_E_ROWS * _DIM) // _WIDE_COLS
_BLOCK_ROWS = 5000
_GRID = _WIDE_ROWS // _BLOCK_ROWS


def _copy_body(in_ref, out_ref):
    out_ref[...] = in_ref[...]


def kernel(embed):
    wide = embed.reshape(_WIDE_ROWS, _WIDE_COLS)
    out = pl.pallas_call(
        _copy_body,
        grid=(_GRID,),
        in_specs=[pl.BlockSpec((_BLOCK_ROWS, _WIDE_COLS), lambda i: (i, 0))],
        out_specs=pl.BlockSpec((_BLOCK_ROWS, _WIDE_COLS), lambda i: (i, 0)),
        out_shape=jax.ShapeDtypeStruct((_WIDE_ROWS, _WIDE_COLS), jnp.float32),
    )(wide)
    return out.reshape(_E_ROWS, _DIM)


# speedup vs baseline: 1.0090x; 1.0090x over previous
import jax
import jax.numpy as jnp
from jax.experimental import pallas as pl
from jax.experimental.pallas import tpu as pltpu

_E_ROWS = 1_000_000
_DIM = 32
_W_COLS = 128
_W_ROWS = (_E_ROWS * _DIM) // _W_COLS
_BLOCK_ROWS = 10000
_GRID = _W_ROWS // _BLOCK_ROWS


def _copy_body(in_ref, out_ref):
    out_ref[...] = in_ref[...]


def kernel(embed):
    wide = embed.reshape(_W_ROWS, _W_COLS)
    out = pl.pallas_call(
        _copy_body,
        grid=(_GRID,),
        in_specs=[pl.BlockSpec((_BLOCK_ROWS, _W_COLS), lambda i: (i, 0))],
        out_specs=pl.BlockSpec((_BLOCK_ROWS, _W_COLS), lambda i: (i, 0)),
        out_shape=jax.ShapeDtypeStruct((_W_ROWS, _W_COLS), jnp.float32),
    )(wide)
    return out.reshape(_E_ROWS, _DIM)


# speedup vs baseline: 1.2267x; 1.2157x over previous
import jax
import jax.numpy as jnp
from jax.experimental import pallas as pl
from jax.experimental.pallas import tpu as pltpu

_E_ROWS = 1_000_000
_DIM = 32
_CHUNK = 10000
_N_CHUNKS = _E_ROWS // _CHUNK
_B = 8
_RW = 4


def _copy_body(in_hbm, out_hbm, bufs, in_sems, out_sems):
    def in_copy(i):
        return pltpu.make_async_copy(
            in_hbm.at[pl.ds(i * _CHUNK, _CHUNK)], bufs.at[i % _B],
            in_sems.at[i % _B],
        )

    def out_copy(i):
        return pltpu.make_async_copy(
            bufs.at[i % _B], out_hbm.at[pl.ds(i * _CHUNK, _CHUNK)],
            out_sems.at[i % _B],
        )

    for i in range(_N_CHUNKS):
        if i >= _B:
            out_copy(i - _B).wait()
        in_copy(i).start()
        if i >= _RW:
            k = i - _RW
            in_copy(k).wait()
            out_copy(k).start()
    for k in range(max(_N_CHUNKS - _RW, 0), _N_CHUNKS):
        in_copy(k).wait()
        out_copy(k).start()
    for k in range(max(_N_CHUNKS - _B, 0), _N_CHUNKS):
        out_copy(k).wait()


def kernel(embed):
    return pl.pallas_call(
        _copy_body,
        out_shape=jax.ShapeDtypeStruct((_E_ROWS, _DIM), jnp.float32),
        in_specs=[pl.BlockSpec(memory_space=pl.ANY)],
        out_specs=pl.BlockSpec(memory_space=pl.ANY),
        scratch_shapes=[
            pltpu.VMEM((_B, _CHUNK, _DIM), jnp.float32),
            pltpu.SemaphoreType.DMA((_B,)),
            pltpu.SemaphoreType.DMA((_B,)),
        ],
    )(embed)
